# baseline (speedup 1.0000x reference)
import functools

import jax
import jax.numpy as jnp
from jax import lax
from jax.experimental import pallas as pl
from jax.experimental.pallas import tpu as pltpu
from jax.experimental.pallas import tpu_sc as plsc

N_CELL = 10000
N_CELL_ACC = 10240
N_TIS = 500
N_TIS_PAD = 512
D = 128
D2 = 256
OUT = 128
EPS = 1e-5

NC = 2
NS = 16
NW = NC * NS
CHUNK = 128

E_CELL = 320000
CPW_C = 79
E_CELL_PAD = NW * CPW_C * CHUNK
E_TIS = 8000
CPW_T = 2
E_TIS_PAD = NW * CPW_T * CHUNK

_MESH = plsc.VectorSubcoreMesh(core_axis_name="c", subcore_axis_name="s")
_LINEAR = pltpu.CompilerParams(use_tc_tiling_on_sc=False)


def _worker_ids():
    c = lax.axis_index("c")
    s = lax.axis_index("s")
    return c, s, s * NC + c


def _deg_body(dst_c, dst_t, ones, zdc, zdt, out_c, out_t,
              di_c, di_t, ones_v, acc_c, acc_t, semi):
    c, s, w = _worker_ids()
    rc = N_CELL_ACC // NS
    rt = N_TIS_PAD // NS
    nc_ = CPW_C * CHUNK
    nt_ = CPW_T * CHUNK
    pltpu.async_copy(dst_c.at[pl.ds(w * nc_, nc_)], di_c, semi)
    pltpu.async_copy(dst_t.at[pl.ds(w * nt_, nt_)], di_t, semi)
    pltpu.sync_copy(ones, ones_v)
    pltpu.sync_copy(zdc.at[pl.ds(s * rc, rc)], acc_c.at[pl.ds(s * rc, rc)])
    pltpu.sync_copy(zdt.at[pl.ds(s * rt, rt)], acc_t.at[pl.ds(s * rt, rt)])
    pltpu.make_async_copy(dst_c.at[pl.ds(w * nc_, nc_)], di_c, semi).wait()
    pltpu.make_async_copy(dst_t.at[pl.ds(w * nt_, nt_)], di_t, semi).wait()
    plsc.subcore_barrier()

    def body_c(j, carry):
        pltpu.sync_copy(ones_v, acc_c.at[di_c.at[pl.ds(j * CHUNK, CHUNK)]],
                        add=True)
        return carry

    lax.fori_loop(0, CPW_C, body_c, 0)
    for j in range(CPW_T):
        pltpu.sync_copy(ones_v, acc_t.at[di_t.at[pl.ds(j * CHUNK, CHUNK)]],
                        add=True)
    plsc.subcore_barrier()
    pltpu.sync_copy(acc_c.at[pl.ds(s * rc, rc)],
                    out_c.at[pl.ds(c * N_CELL_ACC + s * rc, rc)])
    pltpu.sync_copy(acc_t.at[pl.ds(s * rt, rt)],
                    out_t.at[pl.ds(c * N_TIS_PAD + s * rt, rt)])


_deg_call = pl.kernel(
    _deg_body,
    out_type=(
        jax.ShapeDtypeStruct((NC * N_CELL_ACC, 16), jnp.float32),
        jax.ShapeDtypeStruct((NC * N_TIS_PAD, 16), jnp.float32),
    ),
    mesh=_MESH,
    scratch_types=[
        pltpu.VMEM((CPW_C * CHUNK,), jnp.int32),
        pltpu.VMEM((CPW_T * CHUNK,), jnp.int32),
        pltpu.VMEM((CHUNK, 16), jnp.float32),
        pltpu.VMEM_SHARED((N_CELL_ACC, 16), jnp.float32),
        pltpu.VMEM_SHARED((N_TIS_PAD, 16), jnp.float32),
        pltpu.SemaphoreType.DMA,
    ],
    compiler_params=_LINEAR,
)


def _edge_body(n_acc, cpw,
               hs, src2, dst2, zeros, out,
               sidx0, didx0, rows0, sidx1, didx1, rows1, acc, sem0, sem1):
    c, s, w = _worker_ids()
    rpt = n_acc // NS
    pltpu.sync_copy(zeros.at[pl.ds(s * rpt, rpt)], acc.at[pl.ds(s * rpt, rpt)])
    plsc.subcore_barrier()

    def start(j, sidx, didx, rows, sem):
        base = (w + NW * j) * CHUNK
        pltpu.sync_copy(src2.at[pl.ds(base, CHUNK)], sidx)
        pltpu.sync_copy(dst2.at[pl.ds(base, CHUNK)], didx)
        pltpu.async_copy(hs.at[sidx], rows, sem)

    def drain(j, sidx, didx, rows, sem):
        pltpu.make_async_copy(hs.at[sidx], rows, sem).wait()
        pltpu.sync_copy(rows, acc.at[didx], add=True)

    start(0, sidx0, didx0, rows0, sem0)

    def body(i, carry):
        ja = 2 * i

        @pl.when(ja + 1 < cpw)
        def _():
            start(ja + 1, sidx1, didx1, rows1, sem1)
        drain(ja, sidx0, didx0, rows0, sem0)

        @pl.when(ja + 2 < cpw)
        def _():
            start(ja + 2, sidx0, didx0, rows0, sem0)

        @pl.when(ja + 1 < cpw)
        def _():
            drain(ja + 1, sidx1, didx1, rows1, sem1)
        return carry

    lax.fori_loop(0, -(-cpw // 2), body, 0)
    plsc.subcore_barrier()
    pltpu.sync_copy(acc.at[pl.ds(s * rpt, rpt)],
                    out.at[pl.ds(c * n_acc + s * rpt, rpt)])


def _make_edge_call(n_acc, d, cpw):
    return pl.kernel(
        functools.partial(_edge_body, n_acc, cpw),
        out_type=jax.ShapeDtypeStruct((NC * n_acc, d), jnp.float32),
        mesh=_MESH,
        scratch_types=[
            pltpu.VMEM((CHUNK,), jnp.int32),
            pltpu.VMEM((CHUNK,), jnp.int32),
            pltpu.VMEM((CHUNK, d), jnp.float32),
            pltpu.VMEM((CHUNK,), jnp.int32),
            pltpu.VMEM((CHUNK,), jnp.int32),
            pltpu.VMEM((CHUNK, d), jnp.float32),
            pltpu.VMEM_SHARED((n_acc, d), jnp.float32),
            pltpu.SemaphoreType.DMA,
            pltpu.SemaphoreType.DMA,
        ],
        compiler_params=_LINEAR,
    )


_edge_call_c = _make_edge_call(N_CELL_ACC, D, CPW_C)
_edge_call_t = _make_edge_call(N_TIS_PAD, D2, CPW_T)


def _mm0_body(x, w0, out):
    out[...] = jnp.dot(x[...], w0[...], preferred_element_type=jnp.float32)


_mm0_call = pl.pallas_call(
    _mm0_body,
    out_shape=jax.ShapeDtypeStruct((N_CELL, D), jnp.float32),
)


def _scale_body(pc, pt, h0, hs_out, dc_out, dt_out):
    dinv_c = lax.rsqrt(pc[0, :N_CELL, :1] + pc[1, :N_CELL, :1] + 1.0)
    dinv_t = lax.rsqrt(pt[0, :N_TIS, :1] + pt[1, :N_TIS, :1] + 1.0)
    hs_out[...] = h0[...] * dinv_c
    dc_out[...] = dinv_c
    dt_out[...] = dinv_t


_scale_call = pl.pallas_call(
    _scale_body,
    out_shape=(
        jax.ShapeDtypeStruct((N_CELL, D), jnp.float32),
        jax.ShapeDtypeStruct((N_CELL, 1), jnp.float32),
        jax.ShapeDtypeStruct((N_TIS, 1), jnp.float32),
    ),
)


def _norms(x, gnw, gnb, gnm, bng, bnb):
    mean = jnp.mean(x, axis=0, keepdims=True)
    xc = x - gnm * mean
    var = jnp.mean(xc * xc, axis=0, keepdims=True)
    x = gnw * xc * lax.rsqrt(var + EPS) + gnb
    m2 = jnp.mean(x, axis=0, keepdims=True)
    v2 = jnp.mean((x - m2) * (x - m2), axis=0, keepdims=True)
    x = bng * (x - m2) * lax.rsqrt(v2 + EPS) + bnb
    return jnp.maximum(x, 0.0)


def _combine(p, hs_prev, dinv, b, n):
    return dinv[...] * (p[0, :n] + p[1, :n] + hs_prev[...]) + b[...]


def _layer_body(n, p, hs_prev, dinv, b, gnw, gnb, gnm, bng, bnb, w_next, out):
    x = _combine(p, hs_prev, dinv, b, n)
    x = _norms(x, gnw[...], gnb[...], gnm[...], bng[...], bnb[...])
    out[...] = jnp.dot(x, w_next[...],
                       preferred_element_type=jnp.float32) * dinv[...]


_layer_call_c = pl.pallas_call(
    functools.partial(_layer_body, N_CELL),
    out_shape=jax.ShapeDtypeStruct((N_CELL, D), jnp.float32),
)


def _pool_body(p, hs_prev, dinv, b, gnw, gnb, gnm, bng, bnb,
               a_mat, tis_feat, w_t0, dinv_t, out):
    x = _combine(p, hs_prev, dinv, b, N_CELL)
    x = _norms(x, gnw[...], gnb[...], gnm[...], bng[...], bnb[...])
    agg = lax.dot_general(a_mat[0], x, (((0,), (0,)), ((), ())),
                          preferred_element_type=jnp.float32)
    t0 = jnp.concatenate([agg, tis_feat[...]], axis=1)
    out[...] = jnp.dot(t0, w_t0[...],
                       preferred_element_type=jnp.float32) * dinv_t[...]


_pool_call = pl.pallas_call(
    _pool_body,
    out_shape=jax.ShapeDtypeStruct((N_TIS, D2), jnp.float32),
)


_layer_call_t = pl.pallas_call(
    functools.partial(_layer_body, N_TIS),
    out_shape=jax.ShapeDtypeStruct((N_TIS, D2), jnp.float32),
)


def _final_body(p, hs_prev, dinv, b, gnw, gnb, gnm, bng, bnb,
                lin_w, lin_b, out):
    x = _combine(p, hs_prev, dinv, b, N_TIS)
    x = _norms(x, gnw[...], gnb[...], gnm[...], bng[...], bnb[...])
    y = jnp.dot(x, lin_w[...], preferred_element_type=jnp.float32) + lin_b[...]
    out[...] = jnp.mean(y, axis=0, keepdims=True)


_final_call = pl.pallas_call(
    _final_body,
    out_shape=jax.ShapeDtypeStruct((1, OUT), jnp.float32),
)


def _pad_edges(edge, e, e_pad, dummy_row):
    pad = e_pad - e
    src = jnp.concatenate([edge[0], jnp.zeros((pad,), jnp.int32)])
    dst = jnp.concatenate([edge[1], jnp.full((pad,), dummy_row, jnp.int32)])
    return src, dst


def kernel(cell_feat, cell_edge, tissue_feat, tissue_edge, assignment_mat,
           W_c, b_c, gn_w_c, gn_b_c, gn_m_c, bn_g_c, bn_b_c,
           W_t, b_t, gn_w_t, gn_b_t, gn_m_t, bn_g_t, bn_b_t, lin_W, lin_b):
    f32 = jnp.float32
    src_c, dst_c = _pad_edges(cell_edge, E_CELL, E_CELL_PAD, N_CELL)
    src_t, dst_t = _pad_edges(tissue_edge, E_TIS, E_TIS_PAD, N_TIS)

    ones16 = jnp.ones((CHUNK, 16), f32)
    zdc = jnp.zeros((N_CELL_ACC, 16), f32)
    zdt = jnp.zeros((N_TIS_PAD, 16), f32)
    zc = jnp.zeros((N_CELL_ACC, D), f32)
    zt = jnp.zeros((N_TIS_PAD, D2), f32)

    h0 = _mm0_call(cell_feat, W_c[0])
    pc, pt = _deg_call(dst_c, dst_t, ones16, zdc, zdt)
    pc = pc.reshape(NC, N_CELL_ACC, 16)
    pt = pt.reshape(NC, N_TIS_PAD, 16)

    def r1(v):
        return v.reshape(1, -1)

    hs, dinv_c, dinv_t = _scale_call(pc, pt, h0)

    p = _edge_call_c(hs, src_c, dst_c, zc).reshape(NC, N_CELL_ACC, D)
    hs = _layer_call_c(p, hs, dinv_c, r1(b_c[0]), r1(gn_w_c[0]),
                       r1(gn_b_c[0]), r1(gn_m_c[0]), r1(bn_g_c[0]),
                       r1(bn_b_c[0]), W_c[1])

    p = _edge_call_c(hs, src_c, dst_c, zc).reshape(NC, N_CELL_ACC, D)
    ts = _pool_call(p, hs, dinv_c, r1(b_c[1]), r1(gn_w_c[1]), r1(gn_b_c[1]),
                    r1(gn_m_c[1]), r1(bn_g_c[1]), r1(bn_b_c[1]),
                    assignment_mat, tissue_feat, W_t[0], dinv_t)

    p = _edge_call_t(ts, src_t, dst_t, zt).reshape(NC, N_TIS_PAD, D2)
    ts = _layer_call_t(p, ts, dinv_t, r1(b_t[0]), r1(gn_w_t[0]),
                       r1(gn_b_t[0]), r1(gn_m_t[0]), r1(bn_g_t[0]),
                       r1(bn_b_t[0]), W_t[1])

    p = _edge_call_t(ts, src_t, dst_t, zt).reshape(NC, N_TIS_PAD, D2)
    readout = _final_call(p, ts, dinv_t, r1(b_t[1]), r1(gn_w_t[1]),
                          r1(gn_b_t[1]), r1(gn_m_t[1]), r1(bn_g_t[1]),
                          r1(bn_b_t[1]), lin_W, r1(lin_b))
    return readout[:, None, :]


# speedup vs baseline: 1.0006x; 1.0006x over previous
import functools

import jax
import jax.numpy as jnp
from jax import lax
from jax.experimental import pallas as pl
from jax.experimental.pallas import tpu as pltpu
from jax.experimental.pallas import tpu_sc as plsc

N_CELL = 10000
N_CELL_ACC = 10240
N_TIS = 500
N_TIS_PAD = 512
D = 128
D2 = 256
OUT = 128
EPS = 1e-5

NC = 2
NS = 16
NW = NC * NS
CHUNK = 128

E_CELL = 320000
CPW_C = 79
E_CELL_PAD = NW * CPW_C * CHUNK
E_TIS = 8000
CPW_T = 2
E_TIS_PAD = NW * CPW_T * CHUNK

_MESH = plsc.VectorSubcoreMesh(core_axis_name="c", subcore_axis_name="s")
_LINEAR = pltpu.CompilerParams(use_tc_tiling_on_sc=False)


def _worker_ids():
    c = lax.axis_index("c")
    s = lax.axis_index("s")
    return c, s, s * NC + c


def _deg_body(dst_c, dst_t, ones, zdc, zdt, out_c, out_t,
              di_c, di_t, ones_v, acc_c, acc_t, semi):
    c, s, w = _worker_ids()
    rc = N_CELL_ACC // NS
    rt = N_TIS_PAD // NS
    nc_ = CPW_C * CHUNK
    nt_ = CPW_T * CHUNK
    pltpu.async_copy(dst_c.at[pl.ds(w * nc_, nc_)], di_c, semi)
    pltpu.async_copy(dst_t.at[pl.ds(w * nt_, nt_)], di_t, semi)
    pltpu.sync_copy(ones, ones_v)
    pltpu.sync_copy(zdc.at[pl.ds(s * rc, rc)], acc_c.at[pl.ds(s * rc, rc)])
    pltpu.sync_copy(zdt.at[pl.ds(s * rt, rt)], acc_t.at[pl.ds(s * rt, rt)])
    pltpu.make_async_copy(dst_c.at[pl.ds(w * nc_, nc_)], di_c, semi).wait()
    pltpu.make_async_copy(dst_t.at[pl.ds(w * nt_, nt_)], di_t, semi).wait()
    plsc.subcore_barrier()

    def body_c(j, carry):
        pltpu.sync_copy(ones_v, acc_c.at[di_c.at[pl.ds(j * CHUNK, CHUNK)]],
                        add=True)
        return carry

    lax.fori_loop(0, CPW_C, body_c, 0)
    for j in range(CPW_T):
        pltpu.sync_copy(ones_v, acc_t.at[di_t.at[pl.ds(j * CHUNK, CHUNK)]],
                        add=True)
    plsc.subcore_barrier()
    pltpu.sync_copy(acc_c.at[pl.ds(s * rc, rc)],
                    out_c.at[pl.ds(c * N_CELL_ACC + s * rc, rc)])
    pltpu.sync_copy(acc_t.at[pl.ds(s * rt, rt)],
                    out_t.at[pl.ds(c * N_TIS_PAD + s * rt, rt)])


_deg_call = pl.kernel(
    _deg_body,
    out_type=(
        jax.ShapeDtypeStruct((NC * N_CELL_ACC, 16), jnp.float32),
        jax.ShapeDtypeStruct((NC * N_TIS_PAD, 16), jnp.float32),
    ),
    mesh=_MESH,
    scratch_types=[
        pltpu.VMEM((CPW_C * CHUNK,), jnp.int32),
        pltpu.VMEM((CPW_T * CHUNK,), jnp.int32),
        pltpu.VMEM((CHUNK, 16), jnp.float32),
        pltpu.VMEM_SHARED((N_CELL_ACC, 16), jnp.float32),
        pltpu.VMEM_SHARED((N_TIS_PAD, 16), jnp.float32),
        pltpu.SemaphoreType.DMA,
    ],
    compiler_params=_LINEAR,
)


def _edge_body(n_acc, cpw,
               hs, src2, dst2, zeros, out,
               sidx0, didx0, rows0, sidx1, didx1, rows1, acc, sem0, sem1):
    c, s, w = _worker_ids()
    rpt = n_acc // NS
    pltpu.sync_copy(zeros.at[pl.ds(s * rpt, rpt)], acc.at[pl.ds(s * rpt, rpt)])
    plsc.subcore_barrier()

    def start(j, sidx, didx, rows, sem):
        base = (w + NW * j) * CHUNK
        pltpu.sync_copy(src2.at[pl.ds(base, CHUNK)], sidx)
        pltpu.sync_copy(dst2.at[pl.ds(base, CHUNK)], didx)
        pltpu.async_copy(hs.at[sidx], rows, sem)

    def drain(j, sidx, didx, rows, sem):
        pltpu.make_async_copy(hs.at[sidx], rows, sem).wait()
        pltpu.sync_copy(rows, acc.at[didx], add=True)

    start(0, sidx0, didx0, rows0, sem0)

    def body(i, carry):
        ja = 2 * i

        @pl.when(ja + 1 < cpw)
        def _():
            start(ja + 1, sidx1, didx1, rows1, sem1)
        drain(ja, sidx0, didx0, rows0, sem0)

        @pl.when(ja + 2 < cpw)
        def _():
            start(ja + 2, sidx0, didx0, rows0, sem0)

        @pl.when(ja + 1 < cpw)
        def _():
            drain(ja + 1, sidx1, didx1, rows1, sem1)
        return carry

    lax.fori_loop(0, -(-cpw // 2), body, 0)
    plsc.subcore_barrier()
    pltpu.sync_copy(acc.at[pl.ds(s * rpt, rpt)],
                    out.at[pl.ds(c * n_acc + s * rpt, rpt)])


def _make_edge_call(n_acc, d, cpw):
    return pl.kernel(
        functools.partial(_edge_body, n_acc, cpw),
        out_type=jax.ShapeDtypeStruct((NC * n_acc, d), jnp.float32),
        mesh=_MESH,
        scratch_types=[
            pltpu.VMEM((CHUNK,), jnp.int32),
            pltpu.VMEM((CHUNK,), jnp.int32),
            pltpu.VMEM((CHUNK, d), jnp.float32),
            pltpu.VMEM((CHUNK,), jnp.int32),
            pltpu.VMEM((CHUNK,), jnp.int32),
            pltpu.VMEM((CHUNK, d), jnp.float32),
            pltpu.VMEM_SHARED((n_acc, d), jnp.float32),
            pltpu.SemaphoreType.DMA,
            pltpu.SemaphoreType.DMA,
        ],
        compiler_params=_LINEAR,
    )


_edge_call_c = _make_edge_call(N_CELL_ACC, D, CPW_C)
_edge_call_t = _make_edge_call(N_TIS_PAD, D2, CPW_T)


def _mm0_body(x, w0, out):
    out[...] = jnp.dot(x[...], w0[...], preferred_element_type=jnp.float32)


_mm0_call = pl.pallas_call(
    _mm0_body,
    out_shape=jax.ShapeDtypeStruct((N_CELL, D), jnp.float32),
)


def _scale_body(pc, pt, h0, hs_out, dc_out, dt_out):
    dinv_c = lax.rsqrt(pc[0, :N_CELL, :1] + pc[1, :N_CELL, :1] + 1.0)
    dinv_t = lax.rsqrt(pt[0, :N_TIS, :1] + pt[1, :N_TIS, :1] + 1.0)
    hs_out[...] = h0[...] * dinv_c
    dc_out[...] = dinv_c
    dt_out[...] = dinv_t


_scale_call = pl.pallas_call(
    _scale_body,
    out_shape=(
        jax.ShapeDtypeStruct((N_CELL, D), jnp.float32),
        jax.ShapeDtypeStruct((N_CELL, 1), jnp.float32),
        jax.ShapeDtypeStruct((N_TIS, 1), jnp.float32),
    ),
)


def _norms(x, gnw, gnb, gnm, bng, bnb):
    mean = jnp.mean(x, axis=0, keepdims=True)
    xc = x - gnm * mean
    var = jnp.mean(xc * xc, axis=0, keepdims=True)
    x = gnw * xc * lax.rsqrt(var + EPS) + gnb
    m2 = jnp.mean(x, axis=0, keepdims=True)
    v2 = jnp.mean((x - m2) * (x - m2), axis=0, keepdims=True)
    x = bng * (x - m2) * lax.rsqrt(v2 + EPS) + bnb
    return jnp.maximum(x, 0.0)


def _combine(p, hs_prev, dinv, b, n):
    return dinv[...] * (p[0, :n] + p[1, :n] + hs_prev[...]) + b[...]


def _layer_body(n, p, hs_prev, dinv, b, gnw, gnb, gnm, bng, bnb, w_next, out):
    x = _combine(p, hs_prev, dinv, b, n)
    x = _norms(x, gnw[...], gnb[...], gnm[...], bng[...], bnb[...])
    out[...] = jnp.dot(x, w_next[...],
                       preferred_element_type=jnp.float32) * dinv[...]


_layer_call_c = pl.pallas_call(
    functools.partial(_layer_body, N_CELL),
    out_shape=jax.ShapeDtypeStruct((N_CELL, D), jnp.float32),
)


def _pool_body(p, hs_prev, dinv, b, gnw, gnb, gnm, bng, bnb,
               a_mat, tis_feat, w_t0, dinv_t, out):
    x = _combine(p, hs_prev, dinv, b, N_CELL)
    x = _norms(x, gnw[...], gnb[...], gnm[...], bng[...], bnb[...])
    agg = lax.dot_general(a_mat[0], x, (((0,), (0,)), ((), ())),
                          preferred_element_type=jnp.float32)
    t0 = jnp.concatenate([agg, tis_feat[...]], axis=1)
    out[...] = jnp.dot(t0, w_t0[...],
                       preferred_element_type=jnp.float32) * dinv_t[...]


_pool_call = pl.pallas_call(
    _pool_body,
    out_shape=jax.ShapeDtypeStruct((N_TIS, D2), jnp.float32),
)


_layer_call_t = pl.pallas_call(
    functools.partial(_layer_body, N_TIS),
    out_shape=jax.ShapeDtypeStruct((N_TIS, D2), jnp.float32),
)


def _final_body(p, hs_prev, dinv, b, gnw, gnb, gnm, bng, bnb,
                lin_w, lin_b, out):
    x = _combine(p, hs_prev, dinv, b, N_TIS)
    x = _norms(x, gnw[...], gnb[...], gnm[...], bng[...], bnb[...])
    y = jnp.dot(x, lin_w[...], preferred_element_type=jnp.float32) + lin_b[...]
    out[...] = jnp.mean(y, axis=0, keepdims=True)


_final_call = pl.pallas_call(
    _final_body,
    out_shape=jax.ShapeDtypeStruct((1, OUT), jnp.float32),
)


def _pad_edges(edge, e, e_pad, dummy_lo, dummy_n):
    pad = e_pad - e
    dummy = dummy_lo + jnp.arange(pad, dtype=jnp.int32) % dummy_n
    src = jnp.concatenate([edge[0], jnp.zeros((pad,), jnp.int32)])
    dst = jnp.concatenate([edge[1], dummy])
    return src, dst


def kernel(cell_feat, cell_edge, tissue_feat, tissue_edge, assignment_mat,
           W_c, b_c, gn_w_c, gn_b_c, gn_m_c, bn_g_c, bn_b_c,
           W_t, b_t, gn_w_t, gn_b_t, gn_m_t, bn_g_t, bn_b_t, lin_W, lin_b):
    f32 = jnp.float32
    src_c, dst_c = _pad_edges(cell_edge, E_CELL, E_CELL_PAD,
                              N_CELL, N_CELL_ACC - N_CELL)
    src_t, dst_t = _pad_edges(tissue_edge, E_TIS, E_TIS_PAD,
                              N_TIS, N_TIS_PAD - N_TIS)

    ones16 = jnp.ones((CHUNK, 16), f32)
    zdc = jnp.zeros((N_CELL_ACC, 16), f32)
    zdt = jnp.zeros((N_TIS_PAD, 16), f32)
    zc = jnp.zeros((N_CELL_ACC, D), f32)
    zt = jnp.zeros((N_TIS_PAD, D2), f32)

    h0 = _mm0_call(cell_feat, W_c[0])
    pc, pt = _deg_call(dst_c, dst_t, ones16, zdc, zdt)
    pc = pc.reshape(NC, N_CELL_ACC, 16)
    pt = pt.reshape(NC, N_TIS_PAD, 16)

    def r1(v):
        return v.reshape(1, -1)

    hs, dinv_c, dinv_t = _scale_call(pc, pt, h0)

    p = _edge_call_c(hs, src_c, dst_c, zc).reshape(NC, N_CELL_ACC, D)
    hs = _layer_call_c(p, hs, dinv_c, r1(b_c[0]), r1(gn_w_c[0]),
                       r1(gn_b_c[0]), r1(gn_m_c[0]), r1(bn_g_c[0]),
                       r1(bn_b_c[0]), W_c[1])

    p = _edge_call_c(hs, src_c, dst_c, zc).reshape(NC, N_CELL_ACC, D)
    ts = _pool_call(p, hs, dinv_c, r1(b_c[1]), r1(gn_w_c[1]), r1(gn_b_c[1]),
                    r1(gn_m_c[1]), r1(bn_g_c[1]), r1(bn_b_c[1]),
                    assignment_mat, tissue_feat, W_t[0], dinv_t)

    p = _edge_call_t(ts, src_t, dst_t, zt).reshape(NC, N_TIS_PAD, D2)
    ts = _layer_call_t(p, ts, dinv_t, r1(b_t[0]), r1(gn_w_t[0]),
                       r1(gn_b_t[0]), r1(gn_m_t[0]), r1(bn_g_t[0]),
                       r1(bn_b_t[0]), W_t[1])

    p = _edge_call_t(ts, src_t, dst_t, zt).reshape(NC, N_TIS_PAD, D2)
    readout = _final_call(p, ts, dinv_t, r1(b_t[1]), r1(gn_w_t[1]),
                          r1(gn_b_t[1]), r1(gn_m_t[1]), r1(bn_g_t[1]),
                          r1(bn_b_t[1]), lin_W, r1(lin_b))
    return readout[:, None, :]


# speedup vs baseline: 1.3318x; 1.3310x over previous
import functools

import jax
import jax.numpy as jnp
from jax import lax
from jax.experimental import pallas as pl
from jax.experimental.pallas import tpu as pltpu
from jax.experimental.pallas import tpu_sc as plsc

N_CELL = 10000
N_CELL_ACC = 10240
E_CELL = 320000
N_TIS = 500
N_TIS_PAD = 512
E_TIS = 8000
E_TIS_PAD = 8064
D = 128
D2 = 256
OUT = 128
EPS = 1e-5

NC = 2
NS = 16
NW = NC * NS
CHUNK = 128

N_CHUNKS_C = E_CELL // CHUNK
N_ITERS_C = -(-N_CHUNKS_C // NW)
N_CHUNKS_T = E_TIS_PAD // CHUNK
N_ITERS_T = -(-N_CHUNKS_T // NW)

_MESH = plsc.VectorSubcoreMesh(core_axis_name="c", subcore_axis_name="s")
_LINEAR = pltpu.CompilerParams(use_tc_tiling_on_sc=False)


def _worker_ids():
    c = lax.axis_index("c")
    s = lax.axis_index("s")
    return c, s, s * NC + c


def _deg_body(dst_c, dst_t, ones, zc, zt, out_c, out_t,
              didx, ones_v, acc_c, acc_t):
    c, s, w = _worker_ids()
    rc = N_CELL_ACC // NS
    rt = N_TIS_PAD // NS
    pltpu.sync_copy(ones, ones_v)
    pltpu.sync_copy(zc.at[pl.ds(s * rc, rc)], acc_c.at[pl.ds(s * rc, rc)])
    pltpu.sync_copy(zt.at[pl.ds(s * rt, rt)], acc_t.at[pl.ds(s * rt, rt)])
    plsc.subcore_barrier()

    def body_c(j, carry):
        cid = w + NW * j

        @pl.when(cid < N_CHUNKS_C)
        def _():
            pltpu.sync_copy(dst_c.at[pl.ds(cid * CHUNK, CHUNK)], didx)
            pltpu.sync_copy(ones_v, acc_c.at[didx], add=True)
        return carry

    lax.fori_loop(0, N_ITERS_C, body_c, 0)

    def body_t(j, carry):
        cid = w + NW * j

        @pl.when(cid < N_CHUNKS_T)
        def _():
            pltpu.sync_copy(dst_t.at[pl.ds(cid * CHUNK, CHUNK)], didx)
            pltpu.sync_copy(ones_v, acc_t.at[didx], add=True)
        return carry

    lax.fori_loop(0, N_ITERS_T, body_t, 0)
    plsc.subcore_barrier()
    pltpu.sync_copy(acc_c.at[pl.ds(s * rc, rc)],
                    out_c.at[pl.ds(c * N_CELL_ACC + s * rc, rc)])
    pltpu.sync_copy(acc_t.at[pl.ds(s * rt, rt)],
                    out_t.at[pl.ds(c * N_TIS_PAD + s * rt, rt)])


_deg_call = pl.kernel(
    _deg_body,
    out_type=(
        jax.ShapeDtypeStruct((NC * N_CELL_ACC, 16), jnp.float32),
        jax.ShapeDtypeStruct((NC * N_TIS_PAD, 16), jnp.float32),
    ),
    mesh=_MESH,
    scratch_types=[
        pltpu.VMEM((CHUNK,), jnp.int32),
        pltpu.VMEM((CHUNK, 16), jnp.float32),
        pltpu.VMEM_SHARED((N_CELL_ACC, 16), jnp.float32),
        pltpu.VMEM_SHARED((N_TIS_PAD, 16), jnp.float32),
    ],
    compiler_params=_LINEAR,
)


def _edge_body(n_acc, n_chunks, n_iters,
               hs, src, dst, zeros, out,
               sidx0, didx0, rows0, sidx1, didx1, rows1, acc, sem0, sem1):
    c, s, w = _worker_ids()
    rpt = n_acc // NS
    pltpu.sync_copy(zeros.at[pl.ds(s * rpt, rpt)], acc.at[pl.ds(s * rpt, rpt)])
    plsc.subcore_barrier()

    bufs = ((sidx0, didx0, rows0, sem0), (sidx1, didx1, rows1, sem1))

    def start(j, buf):
        sidx, didx, rows, sem = buf
        cid = w + NW * j

        @pl.when(cid < n_chunks)
        def _():
            pltpu.sync_copy(src.at[pl.ds(cid * CHUNK, CHUNK)], sidx)
            pltpu.sync_copy(dst.at[pl.ds(cid * CHUNK, CHUNK)], didx)
            pltpu.async_copy(hs.at[sidx], rows, sem)

    def drain(j, buf):
        sidx, didx, rows, sem = buf
        cid = w + NW * j

        @pl.when(cid < n_chunks)
        def _():
            pltpu.make_async_copy(hs.at[sidx], rows, sem).wait()
            pltpu.sync_copy(rows, acc.at[didx], add=True)

    start(0, bufs[0])

    def body(i, carry):
        ja = 2 * i
        start(ja + 1, bufs[1])
        drain(ja, bufs[0])
        start(ja + 2, bufs[0])
        drain(ja + 1, bufs[1])
        return carry

    lax.fori_loop(0, -(-n_iters // 2), body, 0)
    plsc.subcore_barrier()
    pltpu.sync_copy(acc.at[pl.ds(s * rpt, rpt)],
                    out.at[pl.ds(c * n_acc + s * rpt, rpt)])


def _make_edge_call(n_rows, n_acc, d, n_chunks, n_iters):
    return pl.kernel(
        functools.partial(_edge_body, n_acc, n_chunks, n_iters),
        out_type=jax.ShapeDtypeStruct((NC * n_acc, d), jnp.float32),
        mesh=_MESH,
        scratch_types=[
            pltpu.VMEM((CHUNK,), jnp.int32),
            pltpu.VMEM((CHUNK,), jnp.int32),
            pltpu.VMEM((CHUNK, d), jnp.float32),
            pltpu.VMEM((CHUNK,), jnp.int32),
            pltpu.VMEM((CHUNK,), jnp.int32),
            pltpu.VMEM((CHUNK, d), jnp.float32),
            pltpu.VMEM_SHARED((n_acc, d), jnp.float32),
            pltpu.SemaphoreType.DMA,
            pltpu.SemaphoreType.DMA,
        ],
        compiler_params=_LINEAR,
    )


_edge_call_c = _make_edge_call(N_CELL, N_CELL_ACC, D, N_CHUNKS_C, N_ITERS_C)
_edge_call_t = _make_edge_call(N_TIS, N_TIS_PAD, D2, N_CHUNKS_T, N_ITERS_T)


def _prep_body(pc, pt, x, w0, hs_out, dc_out, dt_out):
    dinv_c = lax.rsqrt(pc[0, :N_CELL, :1] + pc[1, :N_CELL, :1] + 1.0)
    dinv_t = lax.rsqrt(pt[0, :N_TIS, :1] + pt[1, :N_TIS, :1] + 1.0)
    h = jnp.dot(x[...], w0[...], preferred_element_type=jnp.float32)
    hs_out[...] = h * dinv_c
    dc_out[...] = dinv_c
    dt_out[...] = dinv_t


_prep_call = pl.pallas_call(
    _prep_body,
    out_shape=(
        jax.ShapeDtypeStruct((N_CELL, D), jnp.float32),
        jax.ShapeDtypeStruct((N_CELL, 1), jnp.float32),
        jax.ShapeDtypeStruct((N_TIS, 1), jnp.float32),
    ),
)


def _norms(x, gnw, gnb, gnm, bng, bnb):
    mean = jnp.mean(x, axis=0, keepdims=True)
    xc = x - gnm * mean
    var = jnp.mean(xc * xc, axis=0, keepdims=True)
    x = gnw * xc * lax.rsqrt(var + EPS) + gnb
    m2 = jnp.mean(x, axis=0, keepdims=True)
    v2 = jnp.mean((x - m2) * (x - m2), axis=0, keepdims=True)
    x = bng * (x - m2) * lax.rsqrt(v2 + EPS) + bnb
    return jnp.maximum(x, 0.0)


def _combine(p, hs_prev, dinv, b, n):
    return dinv[...] * (p[0, :n] + p[1, :n] + hs_prev[...]) + b[...]


def _layer_body(n, p, hs_prev, dinv, b, gnw, gnb, gnm, bng, bnb, w_next, out):
    x = _combine(p, hs_prev, dinv, b, n)
    x = _norms(x, gnw[...], gnb[...], gnm[...], bng[...], bnb[...])
    out[...] = jnp.dot(x, w_next[...],
                       preferred_element_type=jnp.float32) * dinv[...]


_layer_call_c = pl.pallas_call(
    functools.partial(_layer_body, N_CELL),
    out_shape=jax.ShapeDtypeStruct((N_CELL, D), jnp.float32),
)


def _pool_body(p, hs_prev, dinv, b, gnw, gnb, gnm, bng, bnb,
               a_mat, tis_feat, w_t0, dinv_t, out):
    x = _combine(p, hs_prev, dinv, b, N_CELL)
    x = _norms(x, gnw[...], gnb[...], gnm[...], bng[...], bnb[...])
    agg = lax.dot_general(a_mat[...], x, (((0,), (0,)), ((), ())),
                          preferred_element_type=jnp.float32)
    t0 = jnp.concatenate([agg, tis_feat[...]], axis=1)
    out[...] = jnp.dot(t0, w_t0[...],
                       preferred_element_type=jnp.float32) * dinv_t[...]


_pool_call = pl.pallas_call(
    _pool_body,
    out_shape=jax.ShapeDtypeStruct((N_TIS, D2), jnp.float32),
)


_layer_call_t = pl.pallas_call(
    functools.partial(_layer_body, N_TIS),
    out_shape=jax.ShapeDtypeStruct((N_TIS, D2), jnp.float32),
)


def _final_body(p, hs_prev, dinv, b, gnw, gnb, gnm, bng, bnb,
                lin_w, lin_b, out):
    x = _combine(p, hs_prev, dinv, b, N_TIS)
    x = _norms(x, gnw[...], gnb[...], gnm[...], bng[...], bnb[...])
    y = jnp.dot(x, lin_w[...], preferred_element_type=jnp.float32) + lin_b[...]
    out[...] = jnp.mean(y, axis=0, keepdims=True)


_final_call = pl.pallas_call(
    _final_body,
    out_shape=jax.ShapeDtypeStruct((1, OUT), jnp.float32),
)


def kernel(cell_feat, cell_edge, tissue_feat, tissue_edge, assignment_mat,
           W_c, b_c, gn_w_c, gn_b_c, gn_m_c, bn_g_c, bn_b_c,
           W_t, b_t, gn_w_t, gn_b_t, gn_m_t, bn_g_t, bn_b_t, lin_W, lin_b):
    f32 = jnp.float32
    src_c = cell_edge[0]
    dst_c = cell_edge[1]
    pad = E_TIS_PAD - E_TIS
    src_t = jnp.concatenate([tissue_edge[0], jnp.zeros((pad,), jnp.int32)])
    dst_t = jnp.concatenate(
        [tissue_edge[1], jnp.full((pad,), N_TIS, jnp.int32)])

    ones16 = jnp.ones((CHUNK, 16), f32)
    zdc = jnp.zeros((N_CELL_ACC, 16), f32)
    zdt = jnp.zeros((N_TIS_PAD, 16), f32)
    zc = jnp.zeros((N_CELL_ACC, D), f32)
    zt = jnp.zeros((N_TIS_PAD, D2), f32)

    pc, pt = _deg_call(dst_c, dst_t, ones16, zdc, zdt)
    pc = pc.reshape(NC, N_CELL_ACC, 16)
    pt = pt.reshape(NC, N_TIS_PAD, 16)

    def r1(v):
        return v.reshape(1, -1)

    hs, dinv_c, dinv_t = _prep_call(pc, pt, cell_feat, W_c[0])

    p = _edge_call_c(hs, src_c, dst_c, zc).reshape(NC, N_CELL_ACC, D)
    hs = _layer_call_c(p, hs, dinv_c, r1(b_c[0]), r1(gn_w_c[0]),
                       r1(gn_b_c[0]), r1(gn_m_c[0]), r1(bn_g_c[0]),
                       r1(bn_b_c[0]), W_c[1])

    p = _edge_call_c(hs, src_c, dst_c, zc).reshape(NC, N_CELL_ACC, D)
    ts = _pool_call(p, hs, dinv_c, r1(b_c[1]), r1(gn_w_c[1]), r1(gn_b_c[1]),
                    r1(gn_m_c[1]), r1(bn_g_c[1]), r1(bn_b_c[1]),
                    assignment_mat[0], tissue_feat, W_t[0], dinv_t)

    p = _edge_call_t(ts, src_t, dst_t, zt).reshape(NC, N_TIS_PAD, D2)
    ts = _layer_call_t(p, ts, dinv_t, r1(b_t[0]), r1(gn_w_t[0]),
                       r1(gn_b_t[0]), r1(gn_m_t[0]), r1(bn_g_t[0]),
                       r1(bn_b_t[0]), W_t[1])

    p = _edge_call_t(ts, src_t, dst_t, zt).reshape(NC, N_TIS_PAD, D2)
    readout = _final_call(p, ts, dinv_t, r1(b_t[1]), r1(gn_w_t[1]),
                          r1(gn_b_t[1]), r1(gn_m_t[1]), r1(bn_g_t[1]),
                          r1(bn_b_t[1]), lin_W, r1(lin_b))
    return readout[:, None, :]


# speedup vs baseline: 1.4240x; 1.0692x over previous
import functools

import jax
import jax.numpy as jnp
from jax import lax
from jax.experimental import pallas as pl
from jax.experimental.pallas import tpu as pltpu
from jax.experimental.pallas import tpu_sc as plsc

N_CELL = 10000
N_CELL_ACC = 10240
E_CELL = 320000
N_TIS = 500
N_TIS_PAD = 512
E_TIS = 8000
E_TIS_PAD = 8064
D = 128
D2 = 256
OUT = 128
EPS = 1e-5

NC = 2
NS = 16
NW = NC * NS
CHUNK = 128

N_CHUNKS_C = E_CELL // CHUNK
N_ITERS_C = -(-N_CHUNKS_C // NW)
N_CHUNKS_T = E_TIS_PAD // CHUNK
N_ITERS_T = -(-N_CHUNKS_T // NW)

_MESH = plsc.VectorSubcoreMesh(core_axis_name="c", subcore_axis_name="s")
_LINEAR = pltpu.CompilerParams(use_tc_tiling_on_sc=False)


def _worker_ids():
    c = lax.axis_index("c")
    s = lax.axis_index("s")
    return c, s, s * NC + c


CPW_C = 79
E_CELL_PAD = NW * CPW_C * CHUNK
CPW_T = 2
E_TIS_PAD2 = NW * CPW_T * CHUNK


def _deg_body(dst_c, dst_t, ones, zdc, zdt, out_c, out_t,
              di_c, di_t, ones_v, acc_c, acc_t, semi):
    c, s, w = _worker_ids()
    rc = N_CELL_ACC // NS
    rt = N_TIS_PAD // NS
    nc_ = CPW_C * CHUNK
    nt_ = CPW_T * CHUNK
    pltpu.async_copy(dst_c.at[pl.ds(w * nc_, nc_)], di_c, semi)
    pltpu.async_copy(dst_t.at[pl.ds(w * nt_, nt_)], di_t, semi)
    pltpu.sync_copy(ones, ones_v)
    pltpu.sync_copy(zdc.at[pl.ds(s * rc, rc)], acc_c.at[pl.ds(s * rc, rc)])
    pltpu.sync_copy(zdt.at[pl.ds(s * rt, rt)], acc_t.at[pl.ds(s * rt, rt)])
    pltpu.make_async_copy(dst_c.at[pl.ds(w * nc_, nc_)], di_c, semi).wait()
    pltpu.make_async_copy(dst_t.at[pl.ds(w * nt_, nt_)], di_t, semi).wait()
    plsc.subcore_barrier()

    def body_c(j, carry):
        pltpu.sync_copy(ones_v, acc_c.at[di_c.at[pl.ds(j * CHUNK, CHUNK)]],
                        add=True)
        return carry

    lax.fori_loop(0, CPW_C, body_c, 0)
    for j in range(CPW_T):
        pltpu.sync_copy(ones_v, acc_t.at[di_t.at[pl.ds(j * CHUNK, CHUNK)]],
                        add=True)
    plsc.subcore_barrier()
    pltpu.sync_copy(acc_c.at[pl.ds(s * rc, rc)],
                    out_c.at[pl.ds(c * N_CELL_ACC + s * rc, rc)])
    pltpu.sync_copy(acc_t.at[pl.ds(s * rt, rt)],
                    out_t.at[pl.ds(c * N_TIS_PAD + s * rt, rt)])


_deg_call = pl.kernel(
    _deg_body,
    out_type=(
        jax.ShapeDtypeStruct((NC * N_CELL_ACC, 16), jnp.float32),
        jax.ShapeDtypeStruct((NC * N_TIS_PAD, 16), jnp.float32),
    ),
    mesh=_MESH,
    scratch_types=[
        pltpu.VMEM((CPW_C * CHUNK,), jnp.int32),
        pltpu.VMEM((CPW_T * CHUNK,), jnp.int32),
        pltpu.VMEM((CHUNK, 16), jnp.float32),
        pltpu.VMEM_SHARED((N_CELL_ACC, 16), jnp.float32),
        pltpu.VMEM_SHARED((N_TIS_PAD, 16), jnp.float32),
        pltpu.SemaphoreType.DMA,
    ],
    compiler_params=_LINEAR,
)


def _edge_body(n_acc, n_chunks, n_iters,
               hs, src, dst, zeros, out,
               sidx0, didx0, rows0, sidx1, didx1, rows1, acc, sem0, sem1):
    c, s, w = _worker_ids()
    rpt = n_acc // NS
    pltpu.sync_copy(zeros.at[pl.ds(s * rpt, rpt)], acc.at[pl.ds(s * rpt, rpt)])
    plsc.subcore_barrier()

    bufs = ((sidx0, didx0, rows0, sem0), (sidx1, didx1, rows1, sem1))

    def start(j, buf):
        sidx, didx, rows, sem = buf
        cid = w + NW * j

        @pl.when(cid < n_chunks)
        def _():
            pltpu.sync_copy(src.at[pl.ds(cid * CHUNK, CHUNK)], sidx)
            pltpu.sync_copy(dst.at[pl.ds(cid * CHUNK, CHUNK)], didx)
            pltpu.async_copy(hs.at[sidx], rows, sem)

    def drain(j, buf):
        sidx, didx, rows, sem = buf
        cid = w + NW * j

        @pl.when(cid < n_chunks)
        def _():
            pltpu.make_async_copy(hs.at[sidx], rows, sem).wait()
            pltpu.sync_copy(rows, acc.at[didx], add=True)

    start(0, bufs[0])

    def body(i, carry):
        ja = 2 * i
        start(ja + 1, bufs[1])
        drain(ja, bufs[0])
        start(ja + 2, bufs[0])
        drain(ja + 1, bufs[1])
        return carry

    lax.fori_loop(0, -(-n_iters // 2), body, 0)
    plsc.subcore_barrier()
    pltpu.sync_copy(acc.at[pl.ds(s * rpt, rpt)],
                    out.at[pl.ds(c * n_acc + s * rpt, rpt)])


def _make_edge_call(n_rows, n_acc, d, n_chunks, n_iters):
    return pl.kernel(
        functools.partial(_edge_body, n_acc, n_chunks, n_iters),
        out_type=jax.ShapeDtypeStruct((NC * n_acc, d), jnp.float32),
        mesh=_MESH,
        scratch_types=[
            pltpu.VMEM((CHUNK,), jnp.int32),
            pltpu.VMEM((CHUNK,), jnp.int32),
            pltpu.VMEM((CHUNK, d), jnp.float32),
            pltpu.VMEM((CHUNK,), jnp.int32),
            pltpu.VMEM((CHUNK,), jnp.int32),
            pltpu.VMEM((CHUNK, d), jnp.float32),
            pltpu.VMEM_SHARED((n_acc, d), jnp.float32),
            pltpu.SemaphoreType.DMA,
            pltpu.SemaphoreType.DMA,
        ],
        compiler_params=_LINEAR,
    )


_edge_call_c = _make_edge_call(N_CELL, N_CELL_ACC, D, N_CHUNKS_C, N_ITERS_C)
_edge_call_t = _make_edge_call(N_TIS, N_TIS_PAD, D2, N_CHUNKS_T, N_ITERS_T)


def _prep_body(pc, pt, x, w0, hs_out, dc_out, dt_out):
    dinv_c = lax.rsqrt(pc[0, :N_CELL, :1] + pc[1, :N_CELL, :1] + 1.0)
    dinv_t = lax.rsqrt(pt[0, :N_TIS, :1] + pt[1, :N_TIS, :1] + 1.0)
    h = jnp.dot(x[...], w0[...], preferred_element_type=jnp.float32)
    hs_out[...] = h * dinv_c
    dc_out[...] = dinv_c
    dt_out[...] = dinv_t


_prep_call = pl.pallas_call(
    _prep_body,
    out_shape=(
        jax.ShapeDtypeStruct((N_CELL, D), jnp.float32),
        jax.ShapeDtypeStruct((N_CELL, 1), jnp.float32),
        jax.ShapeDtypeStruct((N_TIS, 1), jnp.float32),
    ),
)


def _norms(x, gnw, gnb, gnm, bng, bnb):
    mean = jnp.mean(x, axis=0, keepdims=True)
    xc = x - gnm * mean
    var = jnp.mean(xc * xc, axis=0, keepdims=True)
    x = gnw * xc * lax.rsqrt(var + EPS) + gnb
    m2 = jnp.mean(x, axis=0, keepdims=True)
    v2 = jnp.mean((x - m2) * (x - m2), axis=0, keepdims=True)
    x = bng * (x - m2) * lax.rsqrt(v2 + EPS) + bnb
    return jnp.maximum(x, 0.0)


def _combine(p, hs_prev, dinv, b, n):
    return dinv[...] * (p[0, :n] + p[1, :n] + hs_prev[...]) + b[...]


def _layer_body(n, p, hs_prev, dinv, b, gnw, gnb, gnm, bng, bnb, w_next, out):
    x = _combine(p, hs_prev, dinv, b, n)
    x = _norms(x, gnw[...], gnb[...], gnm[...], bng[...], bnb[...])
    out[...] = jnp.dot(x, w_next[...],
                       preferred_element_type=jnp.float32) * dinv[...]


_layer_call_c = pl.pallas_call(
    functools.partial(_layer_body, N_CELL),
    out_shape=jax.ShapeDtypeStruct((N_CELL, D), jnp.float32),
)


def _pool_body(p, hs_prev, dinv, b, gnw, gnb, gnm, bng, bnb,
               a_mat, tis_feat, w_t0, dinv_t, out):
    x = _combine(p, hs_prev, dinv, b, N_CELL)
    x = _norms(x, gnw[...], gnb[...], gnm[...], bng[...], bnb[...])
    agg = lax.dot_general(a_mat[...], x, (((0,), (0,)), ((), ())),
                          preferred_element_type=jnp.float32)
    t0 = jnp.concatenate([agg, tis_feat[...]], axis=1)
    out[...] = jnp.dot(t0, w_t0[...],
                       preferred_element_type=jnp.float32) * dinv_t[...]


_pool_call = pl.pallas_call(
    _pool_body,
    out_shape=jax.ShapeDtypeStruct((N_TIS, D2), jnp.float32),
)


_layer_call_t = pl.pallas_call(
    functools.partial(_layer_body, N_TIS),
    out_shape=jax.ShapeDtypeStruct((N_TIS, D2), jnp.float32),
)


def _final_body(p, hs_prev, dinv, b, gnw, gnb, gnm, bng, bnb,
                lin_w, lin_b, out):
    x = _combine(p, hs_prev, dinv, b, N_TIS)
    x = _norms(x, gnw[...], gnb[...], gnm[...], bng[...], bnb[...])
    y = jnp.dot(x, lin_w[...], preferred_element_type=jnp.float32) + lin_b[...]
    out[...] = jnp.mean(y, axis=0, keepdims=True)


_final_call = pl.pallas_call(
    _final_body,
    out_shape=jax.ShapeDtypeStruct((1, OUT), jnp.float32),
)


def kernel(cell_feat, cell_edge, tissue_feat, tissue_edge, assignment_mat,
           W_c, b_c, gn_w_c, gn_b_c, gn_m_c, bn_g_c, bn_b_c,
           W_t, b_t, gn_w_t, gn_b_t, gn_m_t, bn_g_t, bn_b_t, lin_W, lin_b):
    f32 = jnp.float32
    src_c = cell_edge[0]
    dst_c = cell_edge[1]
    pad = E_TIS_PAD - E_TIS
    src_t = jnp.concatenate([tissue_edge[0], jnp.zeros((pad,), jnp.int32)])
    dst_t = jnp.concatenate(
        [tissue_edge[1], jnp.full((pad,), N_TIS, jnp.int32)])

    ones16 = jnp.ones((CHUNK, 16), f32)
    zdc = jnp.zeros((N_CELL_ACC, 16), f32)
    zdt = jnp.zeros((N_TIS_PAD, 16), f32)
    zc = jnp.zeros((N_CELL_ACC, D), f32)
    zt = jnp.zeros((N_TIS_PAD, D2), f32)

    def _pad_dummy(dst, e_pad, lo, n_dummy):
        pad = e_pad - dst.shape[0]
        dummy = lo + jnp.arange(pad, dtype=jnp.int32) % n_dummy
        return jnp.concatenate([dst, dummy])

    dg_c = _pad_dummy(dst_c, E_CELL_PAD, N_CELL, N_CELL_ACC - N_CELL)
    dg_t = _pad_dummy(dst_t, E_TIS_PAD2, N_TIS, N_TIS_PAD - N_TIS)
    pc, pt = _deg_call(dg_c, dg_t, ones16, zdc, zdt)
    pc = pc.reshape(NC, N_CELL_ACC, 16)
    pt = pt.reshape(NC, N_TIS_PAD, 16)

    def r1(v):
        return v.reshape(1, -1)

    hs, dinv_c, dinv_t = _prep_call(pc, pt, cell_feat, W_c[0])

    p = _edge_call_c(hs, src_c, dst_c, zc).reshape(NC, N_CELL_ACC, D)
    hs = _layer_call_c(p, hs, dinv_c, r1(b_c[0]), r1(gn_w_c[0]),
                       r1(gn_b_c[0]), r1(gn_m_c[0]), r1(bn_g_c[0]),
                       r1(bn_b_c[0]), W_c[1])

    p = _edge_call_c(hs, src_c, dst_c, zc).reshape(NC, N_CELL_ACC, D)
    ts = _pool_call(p, hs, dinv_c, r1(b_c[1]), r1(gn_w_c[1]), r1(gn_b_c[1]),
                    r1(gn_m_c[1]), r1(bn_g_c[1]), r1(bn_b_c[1]),
                    assignment_mat[0], tissue_feat, W_t[0], dinv_t)

    p = _edge_call_t(ts, src_t, dst_t, zt).reshape(NC, N_TIS_PAD, D2)
    ts = _layer_call_t(p, ts, dinv_t, r1(b_t[0]), r1(gn_w_t[0]),
                       r1(gn_b_t[0]), r1(gn_m_t[0]), r1(bn_g_t[0]),
                       r1(bn_b_t[0]), W_t[1])

    p = _edge_call_t(ts, src_t, dst_t, zt).reshape(NC, N_TIS_PAD, D2)
    readout = _final_call(p, ts, dinv_t, r1(b_t[1]), r1(gn_w_t[1]),
                          r1(gn_b_t[1]), r1(gn_m_t[1]), r1(bn_g_t[1]),
                          r1(bn_b_t[1]), lin_W, r1(lin_b))
    return readout[:, None, :]


# speedup vs baseline: 1.5928x; 1.1185x over previous
import functools

import jax
import jax.numpy as jnp
from jax import lax
from jax.experimental import pallas as pl
from jax.experimental.pallas import tpu as pltpu
from jax.experimental.pallas import tpu_sc as plsc

N_CELL = 10000
N_CELL_ACC = 10240
E_CELL = 320000
N_TIS = 500
N_TIS_PAD = 512
E_TIS = 8000
E_TIS_PAD = 8064
D = 128
D2 = 256
OUT = 128
EPS = 1e-5

NC = 2
NS = 16
NW = NC * NS
CHUNK = 128

N_CHUNKS_C = E_CELL // CHUNK
N_ITERS_C = -(-N_CHUNKS_C // NW)
N_CHUNKS_T = E_TIS_PAD // CHUNK
N_ITERS_T = -(-N_CHUNKS_T // NW)

_MESH = plsc.VectorSubcoreMesh(core_axis_name="c", subcore_axis_name="s")
_LINEAR = pltpu.CompilerParams(use_tc_tiling_on_sc=False)


def _worker_ids():
    c = lax.axis_index("c")
    s = lax.axis_index("s")
    return c, s, s * NC + c


CPW_C = 79
E_CELL_PAD = NW * CPW_C * CHUNK
CPW_T = 2
E_TIS_PAD2 = NW * CPW_T * CHUNK


def _deg_body(dst_c, dst_t, ones, zdc, zdt, out_c, out_t,
              di_c, di_t, ones_v, acc_c, acc_t, semi):
    c, s, w = _worker_ids()
    rc = N_CELL_ACC // NS
    rt = N_TIS_PAD // NS
    nc_ = CPW_C * CHUNK
    nt_ = CPW_T * CHUNK
    pltpu.async_copy(dst_c.at[pl.ds(w * nc_, nc_)], di_c, semi)
    pltpu.async_copy(dst_t.at[pl.ds(w * nt_, nt_)], di_t, semi)
    pltpu.sync_copy(ones, ones_v)
    pltpu.sync_copy(zdc.at[pl.ds(s * rc, rc)], acc_c.at[pl.ds(s * rc, rc)])
    pltpu.sync_copy(zdt.at[pl.ds(s * rt, rt)], acc_t.at[pl.ds(s * rt, rt)])
    pltpu.make_async_copy(dst_c.at[pl.ds(w * nc_, nc_)], di_c, semi).wait()
    pltpu.make_async_copy(dst_t.at[pl.ds(w * nt_, nt_)], di_t, semi).wait()
    plsc.subcore_barrier()

    def body_c(j, carry):
        pltpu.sync_copy(ones_v, acc_c.at[di_c.at[pl.ds(j * CHUNK, CHUNK)]],
                        add=True)
        return carry

    lax.fori_loop(0, CPW_C, body_c, 0)
    for j in range(CPW_T):
        pltpu.sync_copy(ones_v, acc_t.at[di_t.at[pl.ds(j * CHUNK, CHUNK)]],
                        add=True)
    plsc.subcore_barrier()
    pltpu.sync_copy(acc_c.at[pl.ds(s * rc, rc)],
                    out_c.at[pl.ds(c * N_CELL_ACC + s * rc, rc)])
    pltpu.sync_copy(acc_t.at[pl.ds(s * rt, rt)],
                    out_t.at[pl.ds(c * N_TIS_PAD + s * rt, rt)])


_deg_call = pl.kernel(
    _deg_body,
    out_type=(
        jax.ShapeDtypeStruct((NC * N_CELL_ACC, 16), jnp.float32),
        jax.ShapeDtypeStruct((NC * N_TIS_PAD, 16), jnp.float32),
    ),
    mesh=_MESH,
    scratch_types=[
        pltpu.VMEM((CPW_C * CHUNK,), jnp.int32),
        pltpu.VMEM((CPW_T * CHUNK,), jnp.int32),
        pltpu.VMEM((CHUNK, 16), jnp.float32),
        pltpu.VMEM_SHARED((N_CELL_ACC, 16), jnp.float32),
        pltpu.VMEM_SHARED((N_TIS_PAD, 16), jnp.float32),
        pltpu.SemaphoreType.DMA,
    ],
    compiler_params=_LINEAR,
)


def _edge_body(n_acc, n_chunks, n_iters,
               hs, src, dst, zeros, out,
               sidx0, didx0, rows0, sidx1, didx1, rows1, acc, sem0, sem1):
    c, s, w = _worker_ids()
    rpt = n_acc // NS
    pltpu.sync_copy(zeros.at[pl.ds(s * rpt, rpt)], acc.at[pl.ds(s * rpt, rpt)])
    plsc.subcore_barrier()

    bufs = ((sidx0, didx0, rows0, sem0), (sidx1, didx1, rows1, sem1))

    def start(j, buf):
        sidx, didx, rows, sem = buf
        cid = w + NW * j

        @pl.when(cid < n_chunks)
        def _():
            pltpu.sync_copy(src.at[pl.ds(cid * CHUNK, CHUNK)], sidx)
            pltpu.sync_copy(dst.at[pl.ds(cid * CHUNK, CHUNK)], didx)
            pltpu.async_copy(hs.at[sidx], rows, sem)

    def drain(j, buf):
        sidx, didx, rows, sem = buf
        cid = w + NW * j

        @pl.when(cid < n_chunks)
        def _():
            pltpu.make_async_copy(hs.at[sidx], rows, sem).wait()
            pltpu.sync_copy(rows, acc.at[didx], add=True)

    start(0, bufs[0])

    def body(i, carry):
        ja = 2 * i
        start(ja + 1, bufs[1])
        drain(ja, bufs[0])
        start(ja + 2, bufs[0])
        drain(ja + 1, bufs[1])
        return carry

    lax.fori_loop(0, -(-n_iters // 2), body, 0)
    plsc.subcore_barrier()
    pltpu.sync_copy(acc.at[pl.ds(s * rpt, rpt)],
                    out.at[pl.ds(c * n_acc + s * rpt, rpt)])


def _make_edge_call(n_rows, n_acc, d, n_chunks, n_iters):
    return pl.kernel(
        functools.partial(_edge_body, n_acc, n_chunks, n_iters),
        out_type=jax.ShapeDtypeStruct((NC * n_acc, d), jnp.float32),
        mesh=_MESH,
        scratch_types=[
            pltpu.VMEM((CHUNK,), jnp.int32),
            pltpu.VMEM((CHUNK,), jnp.int32),
            pltpu.VMEM((CHUNK, d), jnp.float32),
            pltpu.VMEM((CHUNK,), jnp.int32),
            pltpu.VMEM((CHUNK,), jnp.int32),
            pltpu.VMEM((CHUNK, d), jnp.float32),
            pltpu.VMEM_SHARED((n_acc, d), jnp.float32),
            pltpu.SemaphoreType.DMA,
            pltpu.SemaphoreType.DMA,
        ],
        compiler_params=_LINEAR,
    )


_edge_call_c = _make_edge_call(N_CELL, N_CELL_ACC, D, N_CHUNKS_C, N_ITERS_C)
_edge_call_t = _make_edge_call(N_TIS, N_TIS_PAD, D2, N_CHUNKS_T, N_ITERS_T)


def _prep_body(pc, pt, x, w0, hs_out, dc_out, dt_out):
    dinv_c = lax.rsqrt(pc[0, :N_CELL, :1] + pc[1, :N_CELL, :1] + 1.0)
    dinv_t = lax.rsqrt(pt[0, :N_TIS, :1] + pt[1, :N_TIS, :1] + 1.0)
    h = jnp.dot(x[...], w0[...], preferred_element_type=jnp.float32)
    hs_out[...] = h * dinv_c
    dc_out[...] = dinv_c
    dt_out[...] = dinv_t


_prep_call = pl.pallas_call(
    _prep_body,
    out_shape=(
        jax.ShapeDtypeStruct((N_CELL, D), jnp.float32),
        jax.ShapeDtypeStruct((N_CELL, 1), jnp.float32),
        jax.ShapeDtypeStruct((N_TIS, 1), jnp.float32),
    ),
)


def _norms(x, gnw, gnb, gnm, bng, bnb):
    mean = jnp.mean(x, axis=0, keepdims=True)
    xc = x - gnm * mean
    var = jnp.mean(xc * xc, axis=0, keepdims=True)
    x = gnw * xc * lax.rsqrt(var + EPS) + gnb
    m2 = jnp.mean(x, axis=0, keepdims=True)
    v2 = jnp.mean((x - m2) * (x - m2), axis=0, keepdims=True)
    x = bng * (x - m2) * lax.rsqrt(v2 + EPS) + bnb
    return jnp.maximum(x, 0.0)


def _combine(p, hs_prev, dinv, b, n):
    return dinv[...] * (p[0, :n] + p[1, :n] + hs_prev[...]) + b[...]


def _layer_body(n, p, hs_prev, dinv, b, gnw, gnb, gnm, bng, bnb, w_next, out):
    x = _combine(p, hs_prev, dinv, b, n)
    x = _norms(x, gnw[...], gnb[...], gnm[...], bng[...], bnb[...])
    out[...] = jnp.dot(x, w_next[...],
                       preferred_element_type=jnp.float32) * dinv[...]


_layer_call_c = pl.pallas_call(
    functools.partial(_layer_body, N_CELL),
    out_shape=jax.ShapeDtypeStruct((N_CELL, D), jnp.float32),
)


def _pool_body(p, hs_prev, dinv, b, gnw, gnb, gnm, bng, bnb,
               a_mat, tis_feat, w_t0, dinv_t, out):
    x = _combine(p, hs_prev, dinv, b, N_CELL)
    x = _norms(x, gnw[...], gnb[...], gnm[...], bng[...], bnb[...])
    agg = lax.dot_general(a_mat[0], x, (((0,), (0,)), ((), ())),
                          preferred_element_type=jnp.float32)
    t0 = jnp.concatenate([agg, tis_feat[...]], axis=1)
    out[...] = jnp.dot(t0, w_t0[...],
                       preferred_element_type=jnp.float32) * dinv_t[...]


_pool_call = pl.pallas_call(
    _pool_body,
    out_shape=jax.ShapeDtypeStruct((N_TIS, D2), jnp.float32),
)


_layer_call_t = pl.pallas_call(
    functools.partial(_layer_body, N_TIS),
    out_shape=jax.ShapeDtypeStruct((N_TIS, D2), jnp.float32),
)


def _final_body(p, hs_prev, dinv, b, gnw, gnb, gnm, bng, bnb,
                lin_w, lin_b, out):
    x = _combine(p, hs_prev, dinv, b, N_TIS)
    x = _norms(x, gnw[...], gnb[...], gnm[...], bng[...], bnb[...])
    y = jnp.dot(x, lin_w[...], preferred_element_type=jnp.float32) + lin_b[...]
    out[...] = jnp.mean(y, axis=0, keepdims=True)


_final_call = pl.pallas_call(
    _final_body,
    out_shape=jax.ShapeDtypeStruct((1, OUT), jnp.float32),
)


def kernel(cell_feat, cell_edge, tissue_feat, tissue_edge, assignment_mat,
           W_c, b_c, gn_w_c, gn_b_c, gn_m_c, bn_g_c, bn_b_c,
           W_t, b_t, gn_w_t, gn_b_t, gn_m_t, bn_g_t, bn_b_t, lin_W, lin_b):
    f32 = jnp.float32
    src_c = cell_edge[0]
    dst_c = cell_edge[1]
    pad = E_TIS_PAD - E_TIS
    src_t = jnp.concatenate([tissue_edge[0], jnp.zeros((pad,), jnp.int32)])
    dst_t = jnp.concatenate(
        [tissue_edge[1], jnp.full((pad,), N_TIS, jnp.int32)])

    ones16 = jnp.ones((CHUNK, 16), f32)
    zdc = jnp.zeros((N_CELL_ACC, 16), f32)
    zdt = jnp.zeros((N_TIS_PAD, 16), f32)
    zc = jnp.zeros((N_CELL_ACC, D), f32)
    zt = jnp.zeros((N_TIS_PAD, D2), f32)

    def _pad_dummy(dst, e_pad, lo, n_dummy):
        pad = e_pad - dst.shape[0]
        dummy = lo + jnp.arange(pad, dtype=jnp.int32) % n_dummy
        return jnp.concatenate([dst, dummy])

    dg_c = _pad_dummy(dst_c, E_CELL_PAD, N_CELL, N_CELL_ACC - N_CELL)
    dg_t = _pad_dummy(dst_t, E_TIS_PAD2, N_TIS, N_TIS_PAD - N_TIS)
    pc, pt = _deg_call(dg_c, dg_t, ones16, zdc, zdt)
    pc = pc.reshape(NC, N_CELL_ACC, 16)
    pt = pt.reshape(NC, N_TIS_PAD, 16)

    def r1(v):
        return v.reshape(1, -1)

    hs, dinv_c, dinv_t = _prep_call(pc, pt, cell_feat, W_c[0])

    p = _edge_call_c(hs, src_c, dst_c, zc).reshape(NC, N_CELL_ACC, D)
    hs = _layer_call_c(p, hs, dinv_c, r1(b_c[0]), r1(gn_w_c[0]),
                       r1(gn_b_c[0]), r1(gn_m_c[0]), r1(bn_g_c[0]),
                       r1(bn_b_c[0]), W_c[1])

    p = _edge_call_c(hs, src_c, dst_c, zc).reshape(NC, N_CELL_ACC, D)
    ts = _pool_call(p, hs, dinv_c, r1(b_c[1]), r1(gn_w_c[1]), r1(gn_b_c[1]),
                    r1(gn_m_c[1]), r1(bn_g_c[1]), r1(bn_b_c[1]),
                    assignment_mat, tissue_feat, W_t[0], dinv_t)

    p = _edge_call_t(ts, src_t, dst_t, zt).reshape(NC, N_TIS_PAD, D2)
    ts = _layer_call_t(p, ts, dinv_t, r1(b_t[0]), r1(gn_w_t[0]),
                       r1(gn_b_t[0]), r1(gn_m_t[0]), r1(bn_g_t[0]),
                       r1(bn_b_t[0]), W_t[1])

    p = _edge_call_t(ts, src_t, dst_t, zt).reshape(NC, N_TIS_PAD, D2)
    readout = _final_call(p, ts, dinv_t, r1(b_t[1]), r1(gn_w_t[1]),
                          r1(gn_b_t[1]), r1(gn_m_t[1]), r1(bn_g_t[1]),
                          r1(bn_b_t[1]), lin_W, r1(lin_b))
    return readout[:, None, :]
